# baseline (speedup 1.0000x reference)
import functools

import jax
import jax.numpy as jnp
from jax import lax
from jax.experimental import pallas as pl
from jax.experimental.pallas import tpu as pltpu
from jax.experimental.pallas import tpu_sc as plsc

N_CORES = 2
N_SUBCORES = 16
LANES = 16
NW = N_CORES * N_SUBCORES
CHUNK = 128


def _lane_shuffle(x, idx):
    dnums = lax.GatherDimensionNumbers(
        offset_dims=(), collapsed_slice_dims=(0,), start_index_map=(0,)
    )
    return lax.gather(
        x, idx[:, None], dnums, slice_sizes=(1,),
        mode=lax.GatherScatterMode.PROMISE_IN_BOUNDS,
    )


def _lane_broadcast_i32(ref, j):
    return plsc.load_gather(ref, [jnp.full((LANES,), j, jnp.int32)])


def _rsqrt_newton(s):
    i = plsc.bitcast(s, jnp.int32)
    y = plsc.bitcast(jnp.int32(0x5F3759DF) - (i >> 1), jnp.float32)
    for _ in range(2):
        y = y * (1.5 - 0.5 * s * y * y)
    return jnp.minimum(y, 1e12)


def _make_sc_kernel(n_pairs, d, chunks_per_w):
    rows_per_w = chunks_per_w * CHUNK
    b_pad = NW * rows_per_w
    n_seg = d // LANES
    mesh = plsc.VectorSubcoreMesh(core_axis_name="c", subcore_axis_name="s")

    @functools.partial(
        pl.kernel,
        out_type=jax.ShapeDtypeStruct((d, b_pad), jnp.float32),
        mesh=mesh,
        compiler_params=pltpu.CompilerParams(
            needs_layout_passes=False, use_tc_tiling_on_sc=True
        ),
        scratch_types=[
            pltpu.VMEM((rows_per_w,), jnp.int32),
            pltpu.VMEM((rows_per_w,), jnp.int32),
            pltpu.VMEM((2, CHUNK, 2 * d), jnp.float32),
            pltpu.VMEM((2, d, CHUNK), jnp.float32),
            pltpu.SemaphoreType.DMA,
            pltpu.SemaphoreType.DMA,
        ],
    )
    def sc_kernel(
        table_hbm, idx_hbm, out_hbm, idx_v, pair_v, buf_v, obuf_v, gsem, osem
    ):
        wid = lax.axis_index("s") * N_CORES + lax.axis_index("c")
        base = wid * rows_per_w
        pltpu.sync_copy(idx_hbm.at[wid], idx_v)

        @plsc.parallel_loop(0, rows_per_w // LANES, unroll=8)
        def pair_body(t):
            pair_v[pl.ds(t * LANES, LANES)] = (
                idx_v[pl.ds(t * LANES, LANES)] >> 1
            )

        def out_slice(j):
            return out_hbm.at[:, pl.ds(base + j * CHUNK, CHUNK)]

        def gather_chunk(j, b):
            return pltpu.async_copy(
                table_hbm.at[pair_v.at[pl.ds(j * CHUNK, CHUNK)]],
                buf_v.at[b],
                gsem,
            )

        gather_chunk(0, 0)

        def chunk_body(j, carry):
            b = j % 2
            nb = (j + 1) % 2

            @pl.when(j >= 2)
            def _():
                pltpu.make_async_copy(
                    obuf_v.at[b], out_slice(j - 2), osem
                ).wait()

            @pl.when(j + 1 < chunks_per_w)
            def _():
                gather_chunk(j + 1, nb)

            pltpu.make_async_copy(
                table_hbm.at[pair_v.at[pl.ds(j * CHUNK, CHUNK)]],
                buf_v.at[b],
                gsem,
            ).wait()

            @plsc.parallel_loop(0, CHUNK // LANES, unroll=1)
            def group_body(g):
                idxv = idx_v[pl.ds(j * CHUNK + g * LANES, LANES)]
                colb = (idxv & 1) * d
                rows = lax.iota(jnp.int32, LANES) + g * LANES
                bsel = jnp.full((LANES,), b, jnp.int32)
                parts = [jnp.zeros((LANES,), jnp.float32) for _ in range(4)]
                for e in range(d):
                    x = plsc.load_gather(buf_v, [bsel, rows, colb + e])
                    parts[e % 4] = parts[e % 4] + x * x
                y = _rsqrt_newton(
                    (parts[0] + parts[1]) + (parts[2] + parts[3])
                )
                for e in range(d):
                    x = plsc.load_gather(buf_v, [bsel, rows, colb + e])
                    obuf_v[b, e, pl.ds(g * LANES, LANES)] = x * y

            pltpu.async_copy(obuf_v.at[b], out_slice(j), osem)
            return carry

        lax.fori_loop(0, chunks_per_w, chunk_body, 0)
        pltpu.make_async_copy(
            obuf_v.at[(chunks_per_w - 2) % 2],
            out_slice(chunks_per_w - 2),
            osem,
        ).wait()
        pltpu.make_async_copy(
            obuf_v.at[(chunks_per_w - 1) % 2],
            out_slice(chunks_per_w - 1),
            osem,
        ).wait()

    return sc_kernel


def kernel(pointsUV, landmarks):
    n_points, d = pointsUV.shape
    b = landmarks.shape[0]
    per_w_chunk = NW * CHUNK
    chunks_per_w = -(-b // per_w_chunk)
    b_pad = chunks_per_w * per_w_chunk
    pad = (jnp.arange(b_pad - b, dtype=jnp.int32) * 8191) % n_points
    idx = jnp.concatenate([landmarks, pad]).reshape(NW, chunks_per_w * CHUNK)
    table2 = pointsUV.reshape(n_points // 2, 2 * d)
    out = _make_sc_kernel(n_points // 2, d, chunks_per_w)(table2, idx)
    return out.T[:b]


# speedup vs baseline: 1.1042x; 1.1042x over previous
import functools

import jax
import jax.numpy as jnp
from jax import lax
from jax.experimental import pallas as pl
from jax.experimental.pallas import tpu as pltpu
from jax.experimental.pallas import tpu_sc as plsc

N_CORES = 2
N_SUBCORES = 16
LANES = 16
NW = N_CORES * N_SUBCORES
CHUNK = 128


def _lane_shuffle(x, idx):
    dnums = lax.GatherDimensionNumbers(
        offset_dims=(), collapsed_slice_dims=(0,), start_index_map=(0,)
    )
    return lax.gather(
        x, idx[:, None], dnums, slice_sizes=(1,),
        mode=lax.GatherScatterMode.PROMISE_IN_BOUNDS,
    )


def _lane_broadcast_i32(ref, j):
    return plsc.load_gather(ref, [jnp.full((LANES,), j, jnp.int32)])


def _rsqrt_newton(s):
    i = plsc.bitcast(s, jnp.int32)
    y = plsc.bitcast(jnp.int32(0x5F3759DF) - (i >> 1), jnp.float32)
    for _ in range(2):
        y = y * (1.5 - 0.5 * s * y * y)
    return jnp.minimum(y, 1e12)


def _make_sc_kernel(n_pairs, d, chunks_per_w):
    rows_per_w = chunks_per_w * CHUNK
    b_pad = NW * rows_per_w
    n_seg = d // LANES
    mesh = plsc.VectorSubcoreMesh(core_axis_name="c", subcore_axis_name="s")

    @functools.partial(
        pl.kernel,
        out_type=jax.ShapeDtypeStruct((d, b_pad), jnp.float32),
        mesh=mesh,
        compiler_params=pltpu.CompilerParams(
            needs_layout_passes=False, use_tc_tiling_on_sc=True
        ),
        scratch_types=[
            pltpu.VMEM((rows_per_w,), jnp.int32),
            pltpu.VMEM((rows_per_w,), jnp.int32),
            pltpu.VMEM((2, CHUNK, 2 * d), jnp.float32),
            pltpu.VMEM((2, d, CHUNK), jnp.float32),
            pltpu.SemaphoreType.DMA,
            pltpu.SemaphoreType.DMA,
        ],
    )
    def sc_kernel(
        table_hbm, idx_hbm, out_hbm, idx_v, pair_v, buf_v, obuf_v, gsem, osem
    ):
        wid = lax.axis_index("s") * N_CORES + lax.axis_index("c")
        base = wid * rows_per_w
        pltpu.sync_copy(idx_hbm.at[wid], idx_v)

        @plsc.parallel_loop(0, rows_per_w // LANES, unroll=8)
        def pair_body(t):
            pair_v[pl.ds(t * LANES, LANES)] = (
                idx_v[pl.ds(t * LANES, LANES)] >> 1
            )

        def out_slice(j):
            return out_hbm.at[:, pl.ds(base + j * CHUNK, CHUNK)]

        def gather_chunk(j, b):
            return pltpu.async_copy(
                table_hbm.at[pair_v.at[pl.ds(j * CHUNK, CHUNK)]],
                buf_v.at[b],
                gsem,
            )

        gather_chunk(0, 0)

        def chunk_body(j, carry):
            b = j % 2
            nb = (j + 1) % 2

            @pl.when(j >= 2)
            def _():
                pltpu.make_async_copy(
                    obuf_v.at[b], out_slice(j - 2), osem
                ).wait()

            @pl.when(j + 1 < chunks_per_w)
            def _():
                gather_chunk(j + 1, nb)

            pltpu.make_async_copy(
                table_hbm.at[pair_v.at[pl.ds(j * CHUNK, CHUNK)]],
                buf_v.at[b],
                gsem,
            ).wait()

            @plsc.parallel_loop(0, CHUNK // LANES, unroll=1)
            def group_body(g):
                idxv = idx_v[pl.ds(j * CHUNK + g * LANES, LANES)]
                colb = (idxv & 1) * d
                lanes = lax.iota(jnp.int32, LANES)
                rows = lanes + g * LANES
                bsel = jnp.full((LANES,), b, jnp.int32)
                parts = [jnp.zeros((LANES,), jnp.float32) for _ in range(4)]
                for e in range(d):
                    eskew = (lanes + e) & (d - 1)
                    x = plsc.load_gather(buf_v, [bsel, rows, colb + eskew])
                    parts[e % 4] = parts[e % 4] + x * x
                y = _rsqrt_newton(
                    (parts[0] + parts[1]) + (parts[2] + parts[3])
                )
                for e in range(d):
                    eskew = (lanes + e) & (d - 1)
                    x = plsc.load_gather(buf_v, [bsel, rows, colb + eskew])
                    plsc.store_scatter(obuf_v, [bsel, eskew, rows], x * y)

            pltpu.async_copy(obuf_v.at[b], out_slice(j), osem)
            return carry

        lax.fori_loop(0, chunks_per_w, chunk_body, 0)
        pltpu.make_async_copy(
            obuf_v.at[(chunks_per_w - 2) % 2],
            out_slice(chunks_per_w - 2),
            osem,
        ).wait()
        pltpu.make_async_copy(
            obuf_v.at[(chunks_per_w - 1) % 2],
            out_slice(chunks_per_w - 1),
            osem,
        ).wait()

    return sc_kernel


def kernel(pointsUV, landmarks):
    n_points, d = pointsUV.shape
    b = landmarks.shape[0]
    per_w_chunk = NW * CHUNK
    chunks_per_w = -(-b // per_w_chunk)
    b_pad = chunks_per_w * per_w_chunk
    pad = (jnp.arange(b_pad - b, dtype=jnp.int32) * 8191) % n_points
    idx = jnp.concatenate([landmarks, pad]).reshape(NW, chunks_per_w * CHUNK)
    table2 = pointsUV.reshape(n_points // 2, 2 * d)
    out = _make_sc_kernel(n_points // 2, d, chunks_per_w)(table2, idx)
    return out.T[:b]


# speedup vs baseline: 1.2113x; 1.0970x over previous
import functools

import jax
import jax.numpy as jnp
from jax import lax
from jax.experimental import pallas as pl
from jax.experimental.pallas import tpu as pltpu
from jax.experimental.pallas import tpu_sc as plsc

N_CORES = 2
N_SUBCORES = 16
LANES = 16
NW = N_CORES * N_SUBCORES
CHUNK = 128


def _lane_shuffle(x, idx):
    dnums = lax.GatherDimensionNumbers(
        offset_dims=(), collapsed_slice_dims=(0,), start_index_map=(0,)
    )
    return lax.gather(
        x, idx[:, None], dnums, slice_sizes=(1,),
        mode=lax.GatherScatterMode.PROMISE_IN_BOUNDS,
    )


def _lane_broadcast_i32(ref, j):
    return plsc.load_gather(ref, [jnp.full((LANES,), j, jnp.int32)])


def _rsqrt_newton(s):
    i = plsc.bitcast(s, jnp.int32)
    y = plsc.bitcast(jnp.int32(0x5F3759DF) - (i >> 1), jnp.float32)
    for _ in range(2):
        y = y * (1.5 - 0.5 * s * y * y)
    return jnp.minimum(y, 1e12)


def _make_sc_kernel(n_pairs, d, chunks_per_w):
    rows_per_w = chunks_per_w * CHUNK
    b_pad = NW * rows_per_w
    n_seg = d // LANES
    mesh = plsc.VectorSubcoreMesh(core_axis_name="c", subcore_axis_name="s")

    @functools.partial(
        pl.kernel,
        out_type=jax.ShapeDtypeStruct((b_pad, d), jnp.float32),
        mesh=mesh,
        compiler_params=pltpu.CompilerParams(
            needs_layout_passes=False, use_tc_tiling_on_sc=True
        ),
        scratch_types=[
            pltpu.VMEM((rows_per_w,), jnp.int32),
            pltpu.VMEM((rows_per_w,), jnp.int32),
            pltpu.VMEM((2, CHUNK, 2 * d), jnp.float32),
            pltpu.VMEM((2, CHUNK, d), jnp.float32),
            pltpu.SemaphoreType.DMA,
            pltpu.SemaphoreType.DMA,
        ],
    )
    def sc_kernel(
        table_hbm, idx_hbm, out_hbm, idx_v, pair_v, buf_v, obuf_v, gsem, osem
    ):
        wid = lax.axis_index("s") * N_CORES + lax.axis_index("c")
        base = wid * rows_per_w
        pltpu.sync_copy(idx_hbm.at[wid], idx_v)

        @plsc.parallel_loop(0, rows_per_w // LANES, unroll=8)
        def pair_body(t):
            pair_v[pl.ds(t * LANES, LANES)] = (
                idx_v[pl.ds(t * LANES, LANES)] >> 1
            )

        def out_slice(j):
            return out_hbm.at[pl.ds(base + j * CHUNK, CHUNK)]

        def gather_chunk(j, b):
            return pltpu.async_copy(
                table_hbm.at[pair_v.at[pl.ds(j * CHUNK, CHUNK)]],
                buf_v.at[b],
                gsem,
            )

        gather_chunk(0, 0)

        def chunk_body(j, carry):
            b = j % 2
            nb = (j + 1) % 2

            @pl.when(j >= 2)
            def _():
                pltpu.make_async_copy(
                    obuf_v.at[b], out_slice(j - 2), osem
                ).wait()

            @pl.when(j + 1 < chunks_per_w)
            def _():
                gather_chunk(j + 1, nb)

            pltpu.make_async_copy(
                table_hbm.at[pair_v.at[pl.ds(j * CHUNK, CHUNK)]],
                buf_v.at[b],
                gsem,
            ).wait()

            @plsc.parallel_loop(0, CHUNK, unroll=4)
            def row_body(r):
                xs = [
                    buf_v[b, r, pl.ds(k * LANES, LANES)]
                    for k in range(2 * n_seg)
                ]
                odd = (_lane_broadcast_i32(idx_v, j * CHUNK + r) & 1) != 0
                hs = [
                    jnp.where(odd, xs[k + n_seg], xs[k]) for k in range(n_seg)
                ]
                sq = hs[0] * hs[0]
                for k in range(1, n_seg):
                    sq = sq + hs[k] * hs[k]
                for k in (1, 2, 4, 8):
                    sq = sq + _lane_shuffle(
                        sq, jnp.bitwise_xor(lax.iota(jnp.int32, LANES), k)
                    )
                y = _rsqrt_newton(sq)
                for k in range(n_seg):
                    obuf_v[b, r, pl.ds(k * LANES, LANES)] = hs[k] * y

            pltpu.async_copy(obuf_v.at[b], out_slice(j), osem)
            return carry

        lax.fori_loop(0, chunks_per_w, chunk_body, 0)
        pltpu.make_async_copy(
            obuf_v.at[(chunks_per_w - 2) % 2],
            out_slice(chunks_per_w - 2),
            osem,
        ).wait()
        pltpu.make_async_copy(
            obuf_v.at[(chunks_per_w - 1) % 2],
            out_slice(chunks_per_w - 1),
            osem,
        ).wait()

    return sc_kernel


def kernel(pointsUV, landmarks):
    n_points, d = pointsUV.shape
    b = landmarks.shape[0]
    per_w_chunk = NW * CHUNK
    chunks_per_w = -(-b // per_w_chunk)
    b_pad = chunks_per_w * per_w_chunk
    pad = (jnp.arange(b_pad - b, dtype=jnp.int32) * 8191) % n_points
    idx = jnp.concatenate([landmarks, pad]).reshape(NW, chunks_per_w * CHUNK)
    table2 = pointsUV.reshape(n_points // 2, 2 * d)
    out = _make_sc_kernel(n_points // 2, d, chunks_per_w)(table2, idx)
    return out[:b]


# speedup vs baseline: 2.3196x; 1.9151x over previous
import functools

import jax
import jax.numpy as jnp
from jax import lax
from jax.experimental import pallas as pl
from jax.experimental.pallas import tpu as pltpu
from jax.experimental.pallas import tpu_sc as plsc

N_CORES = 2
N_SUBCORES = 16
LANES = 16
NW = N_CORES * N_SUBCORES
CHUNK = 128


def _lane_shuffle(x, idx):
    dnums = lax.GatherDimensionNumbers(
        offset_dims=(), collapsed_slice_dims=(0,), start_index_map=(0,)
    )
    return lax.gather(
        x, idx[:, None], dnums, slice_sizes=(1,),
        mode=lax.GatherScatterMode.PROMISE_IN_BOUNDS,
    )


def _lane_broadcast_i32(ref, j):
    return plsc.load_gather(ref, [jnp.full((LANES,), j, jnp.int32)])


def _rsqrt_newton(s):
    i = plsc.bitcast(s, jnp.int32)
    y = plsc.bitcast(jnp.int32(0x5F3759DF) - (i >> 1), jnp.float32)
    for _ in range(2):
        y = y * (1.5 - 0.5 * s * y * y)
    return jnp.minimum(y, 1e12)


def _make_tc_repack(n_points, d, block_i=8192):
    grid = -(-n_points // block_i)

    half = block_i // 2

    def body(t_ref, o_ref):
        xt = t_ref[...].T
        o_ref[:, 0:d] = xt[0:half, :]
        o_ref[:, d:2 * d] = xt[half:, :]

    return pl.pallas_call(
        body,
        grid=(grid,),
        in_specs=[pl.BlockSpec((d, block_i), lambda g: (0, g))],
        out_specs=pl.BlockSpec((half, 2 * d), lambda g: (g, 0)),
        out_shape=jax.ShapeDtypeStruct((grid * half, 2 * d), jnp.float32),
    )


def _make_sc_kernel(n_pairs, d, chunks_per_w):
    rows_per_w = chunks_per_w * CHUNK
    b_pad = NW * rows_per_w
    n_seg = d // LANES
    mesh = plsc.VectorSubcoreMesh(core_axis_name="c", subcore_axis_name="s")

    @functools.partial(
        pl.kernel,
        out_type=jax.ShapeDtypeStruct((b_pad, d), jnp.float32),
        mesh=mesh,
        compiler_params=pltpu.CompilerParams(
            needs_layout_passes=False, use_tc_tiling_on_sc=True
        ),
        scratch_types=[
            pltpu.VMEM((rows_per_w,), jnp.int32),
            pltpu.VMEM((rows_per_w,), jnp.int32),
            pltpu.VMEM((2, CHUNK, 2 * d), jnp.float32),
            pltpu.VMEM((2, CHUNK, d), jnp.float32),
            pltpu.SemaphoreType.DMA,
            pltpu.SemaphoreType.DMA,
        ],
    )
    def sc_kernel(
        table_hbm, idx_hbm, out_hbm, idx_v, pair_v, buf_v, obuf_v, gsem, osem
    ):
        wid = lax.axis_index("s") * N_CORES + lax.axis_index("c")
        base = wid * rows_per_w
        pltpu.sync_copy(idx_hbm.at[wid], idx_v)

        @plsc.parallel_loop(0, rows_per_w // LANES, unroll=8)
        def pair_body(t):
            v = idx_v[pl.ds(t * LANES, LANES)]
            pair_v[pl.ds(t * LANES, LANES)] = ((v >> 13) << 12) | (v & 4095)

        def out_slice(j):
            return out_hbm.at[pl.ds(base + j * CHUNK, CHUNK)]

        def gather_chunk(j, b):
            return pltpu.async_copy(
                table_hbm.at[pair_v.at[pl.ds(j * CHUNK, CHUNK)]],
                buf_v.at[b],
                gsem,
            )

        gather_chunk(0, 0)

        def chunk_body(j, carry):
            b = j % 2
            nb = (j + 1) % 2

            @pl.when(j >= 2)
            def _():
                pltpu.make_async_copy(
                    obuf_v.at[b], out_slice(j - 2), osem
                ).wait()

            @pl.when(j + 1 < chunks_per_w)
            def _():
                gather_chunk(j + 1, nb)

            pltpu.make_async_copy(
                table_hbm.at[pair_v.at[pl.ds(j * CHUNK, CHUNK)]],
                buf_v.at[b],
                gsem,
            ).wait()

            @plsc.parallel_loop(0, CHUNK, unroll=4)
            def row_body(r):
                xs = [
                    buf_v[b, r, pl.ds(k * LANES, LANES)]
                    for k in range(2 * n_seg)
                ]
                odd = (
                    (_lane_broadcast_i32(idx_v, j * CHUNK + r) >> 12) & 1
                ) != 0
                hs = [
                    jnp.where(odd, xs[k + n_seg], xs[k]) for k in range(n_seg)
                ]
                sq = hs[0] * hs[0]
                for k in range(1, n_seg):
                    sq = sq + hs[k] * hs[k]
                for k in (1, 2, 4, 8):
                    sq = sq + _lane_shuffle(
                        sq, jnp.bitwise_xor(lax.iota(jnp.int32, LANES), k)
                    )
                y = _rsqrt_newton(sq)
                for k in range(n_seg):
                    obuf_v[b, r, pl.ds(k * LANES, LANES)] = hs[k] * y

            pltpu.async_copy(obuf_v.at[b], out_slice(j), osem)
            return carry

        lax.fori_loop(0, chunks_per_w, chunk_body, 0)
        pltpu.make_async_copy(
            obuf_v.at[(chunks_per_w - 2) % 2],
            out_slice(chunks_per_w - 2),
            osem,
        ).wait()
        pltpu.make_async_copy(
            obuf_v.at[(chunks_per_w - 1) % 2],
            out_slice(chunks_per_w - 1),
            osem,
        ).wait()

    return sc_kernel


def kernel(pointsUV, landmarks):
    n_points, d = pointsUV.shape
    b = landmarks.shape[0]
    per_w_chunk = NW * CHUNK
    chunks_per_w = -(-b // per_w_chunk)
    b_pad = chunks_per_w * per_w_chunk
    pad = (jnp.arange(b_pad - b, dtype=jnp.int32) * 8191) % n_points
    idx = jnp.concatenate([landmarks, pad]).reshape(NW, chunks_per_w * CHUNK)
    table2 = _make_tc_repack(n_points, d)(pointsUV.T)
    out = _make_sc_kernel(table2.shape[0], d, chunks_per_w)(table2, idx)
    return out[:b]


# speedup vs baseline: 2.5379x; 1.0941x over previous
import functools

import jax
import jax.numpy as jnp
from jax import lax
from jax.experimental import pallas as pl
from jax.experimental.pallas import tpu as pltpu
from jax.experimental.pallas import tpu_sc as plsc

N_CORES = 2
N_SUBCORES = 16
LANES = 16
NW = N_CORES * N_SUBCORES
CHUNK = 128


def _lane_shuffle(x, idx):
    dnums = lax.GatherDimensionNumbers(
        offset_dims=(), collapsed_slice_dims=(0,), start_index_map=(0,)
    )
    return lax.gather(
        x, idx[:, None], dnums, slice_sizes=(1,),
        mode=lax.GatherScatterMode.PROMISE_IN_BOUNDS,
    )


def _lane_broadcast_i32(ref, j):
    return plsc.load_gather(ref, [jnp.full((LANES,), j, jnp.int32)])


def _rsqrt_newton(s):
    i = plsc.bitcast(s, jnp.int32)
    y = plsc.bitcast(jnp.int32(0x5F3759DF) - (i >> 1), jnp.float32)
    for _ in range(2):
        y = y * (1.5 - 0.5 * s * y * y)
    return jnp.minimum(y, 1e12)


BLOCK_I = 16384


def _make_tc_repack(n_points, d, block_i=BLOCK_I):
    grid = -(-n_points // block_i)

    half = block_i // 2

    def body(t_ref, o_ref):
        xt = t_ref[...].T
        o_ref[:, 0:d] = xt[0:half, :]
        o_ref[:, d:2 * d] = xt[half:, :]

    return pl.pallas_call(
        body,
        grid=(grid,),
        in_specs=[pl.BlockSpec((d, block_i), lambda g: (0, g))],
        out_specs=pl.BlockSpec((half, 2 * d), lambda g: (g, 0)),
        out_shape=jax.ShapeDtypeStruct((grid * half, 2 * d), jnp.float32),
    )


def _make_sc_kernel(n_pairs, d, chunks_per_w):
    rows_per_w = chunks_per_w * CHUNK
    b_pad = NW * rows_per_w
    n_seg = d // LANES
    mesh = plsc.VectorSubcoreMesh(core_axis_name="c", subcore_axis_name="s")

    @functools.partial(
        pl.kernel,
        out_type=jax.ShapeDtypeStruct((b_pad, d), jnp.float32),
        mesh=mesh,
        compiler_params=pltpu.CompilerParams(
            needs_layout_passes=False, use_tc_tiling_on_sc=True
        ),
        scratch_types=[
            pltpu.VMEM((rows_per_w,), jnp.int32),
            pltpu.VMEM((rows_per_w,), jnp.int32),
            pltpu.VMEM((2, CHUNK, 2 * d), jnp.float32),
            pltpu.VMEM((2, CHUNK, d), jnp.float32),
            pltpu.SemaphoreType.DMA,
            pltpu.SemaphoreType.DMA,
        ],
    )
    def sc_kernel(
        table_hbm, idx_hbm, out_hbm, idx_v, pair_v, buf_v, obuf_v, gsem, osem
    ):
        wid = lax.axis_index("s") * N_CORES + lax.axis_index("c")
        base = wid * rows_per_w
        pltpu.sync_copy(idx_hbm.at[wid], idx_v)

        sb = BLOCK_I.bit_length() - 1
        sh = sb - 1
        mh = (BLOCK_I // 2) - 1

        @plsc.parallel_loop(0, rows_per_w // LANES, unroll=8)
        def pair_body(t):
            v = idx_v[pl.ds(t * LANES, LANES)]
            pair_v[pl.ds(t * LANES, LANES)] = ((v >> sb) << sh) | (v & mh)

        def out_slice(j):
            return out_hbm.at[pl.ds(base + j * CHUNK, CHUNK)]

        def gather_chunk(j, b):
            return pltpu.async_copy(
                table_hbm.at[pair_v.at[pl.ds(j * CHUNK, CHUNK)]],
                buf_v.at[b],
                gsem,
            )

        gather_chunk(0, 0)

        def chunk_body(j, carry):
            b = j % 2
            nb = (j + 1) % 2

            @pl.when(j >= 2)
            def _():
                pltpu.make_async_copy(
                    obuf_v.at[b], out_slice(j - 2), osem
                ).wait()

            @pl.when(j + 1 < chunks_per_w)
            def _():
                gather_chunk(j + 1, nb)

            pltpu.make_async_copy(
                table_hbm.at[pair_v.at[pl.ds(j * CHUNK, CHUNK)]],
                buf_v.at[b],
                gsem,
            ).wait()

            @plsc.parallel_loop(0, CHUNK, unroll=4)
            def row_body(r):
                xs = [
                    buf_v[b, r, pl.ds(k * LANES, LANES)]
                    for k in range(2 * n_seg)
                ]
                odd = (
                    (
                        _lane_broadcast_i32(idx_v, j * CHUNK + r)
                        >> (BLOCK_I.bit_length() - 2)
                    )
                    & 1
                ) != 0
                hs = [
                    jnp.where(odd, xs[k + n_seg], xs[k]) for k in range(n_seg)
                ]
                sq = hs[0] * hs[0]
                for k in range(1, n_seg):
                    sq = sq + hs[k] * hs[k]
                for k in (1, 2, 4, 8):
                    sq = sq + _lane_shuffle(
                        sq, jnp.bitwise_xor(lax.iota(jnp.int32, LANES), k)
                    )
                y = _rsqrt_newton(sq)
                for k in range(n_seg):
                    obuf_v[b, r, pl.ds(k * LANES, LANES)] = hs[k] * y

            pltpu.async_copy(obuf_v.at[b], out_slice(j), osem)
            return carry

        lax.fori_loop(0, chunks_per_w, chunk_body, 0)
        pltpu.make_async_copy(
            obuf_v.at[(chunks_per_w - 2) % 2],
            out_slice(chunks_per_w - 2),
            osem,
        ).wait()
        pltpu.make_async_copy(
            obuf_v.at[(chunks_per_w - 1) % 2],
            out_slice(chunks_per_w - 1),
            osem,
        ).wait()

    return sc_kernel


def kernel(pointsUV, landmarks):
    n_points, d = pointsUV.shape
    b = landmarks.shape[0]
    per_w_chunk = NW * CHUNK
    chunks_per_w = -(-b // per_w_chunk)
    b_pad = chunks_per_w * per_w_chunk
    pad = (jnp.arange(b_pad - b, dtype=jnp.int32) * 8191) % n_points
    idx = jnp.concatenate([landmarks, pad]).reshape(NW, chunks_per_w * CHUNK)
    table2 = _make_tc_repack(n_points, d)(pointsUV.T)
    out = _make_sc_kernel(table2.shape[0], d, chunks_per_w)(table2, idx)
    return out[:b]


# speedup vs baseline: 2.6442x; 1.0419x over previous
import functools

import jax
import jax.numpy as jnp
from jax import lax
from jax.experimental import pallas as pl
from jax.experimental.pallas import tpu as pltpu
from jax.experimental.pallas import tpu_sc as plsc

N_CORES = 2
N_SUBCORES = 16
LANES = 16
NW = N_CORES * N_SUBCORES
CHUNK = 128


def _lane_shuffle(x, idx):
    dnums = lax.GatherDimensionNumbers(
        offset_dims=(), collapsed_slice_dims=(0,), start_index_map=(0,)
    )
    return lax.gather(
        x, idx[:, None], dnums, slice_sizes=(1,),
        mode=lax.GatherScatterMode.PROMISE_IN_BOUNDS,
    )


def _lane_broadcast_i32(ref, j):
    return plsc.load_gather(ref, [jnp.full((LANES,), j, jnp.int32)])


def _rsqrt_newton(s):
    i = plsc.bitcast(s, jnp.int32)
    y = plsc.bitcast(jnp.int32(0x5F3759DF) - (i >> 1), jnp.float32)
    for _ in range(2):
        y = y * (1.5 - 0.5 * s * y * y)
    return jnp.minimum(y, 1e12)


BLOCK_I = 32768


def _make_tc_repack(n_points, d, block_i=BLOCK_I):
    grid = -(-n_points // block_i)

    half = block_i // 2

    def body(t_ref, o_ref):
        xt = t_ref[...].T
        o_ref[:, 0:d] = xt[0:half, :]
        o_ref[:, d:2 * d] = xt[half:, :]

    return pl.pallas_call(
        body,
        grid=(grid,),
        in_specs=[pl.BlockSpec((d, block_i), lambda g: (0, g))],
        out_specs=pl.BlockSpec((half, 2 * d), lambda g: (g, 0)),
        out_shape=jax.ShapeDtypeStruct((grid * half, 2 * d), jnp.float32),
    )


def _make_sc_kernel(n_pairs, d, chunks_per_w):
    rows_per_w = chunks_per_w * CHUNK
    b_pad = NW * rows_per_w
    n_seg = d // LANES
    mesh = plsc.VectorSubcoreMesh(core_axis_name="c", subcore_axis_name="s")

    @functools.partial(
        pl.kernel,
        out_type=jax.ShapeDtypeStruct((b_pad, d), jnp.float32),
        mesh=mesh,
        compiler_params=pltpu.CompilerParams(
            needs_layout_passes=False, use_tc_tiling_on_sc=True
        ),
        scratch_types=[
            pltpu.VMEM((rows_per_w,), jnp.int32),
            pltpu.VMEM((rows_per_w,), jnp.int32),
            pltpu.VMEM((2, CHUNK, 2 * d), jnp.float32),
            pltpu.VMEM((2, CHUNK, d), jnp.float32),
            pltpu.SemaphoreType.DMA,
            pltpu.SemaphoreType.DMA,
        ],
    )
    def sc_kernel(
        table_hbm, idx_hbm, out_hbm, idx_v, pair_v, buf_v, obuf_v, gsem, osem
    ):
        wid = lax.axis_index("s") * N_CORES + lax.axis_index("c")
        base = wid * rows_per_w
        pltpu.sync_copy(idx_hbm.at[wid], idx_v)

        sb = BLOCK_I.bit_length() - 1
        sh = sb - 1
        mh = (BLOCK_I // 2) - 1

        @plsc.parallel_loop(0, rows_per_w // LANES, unroll=8)
        def pair_body(t):
            v = idx_v[pl.ds(t * LANES, LANES)]
            pair_v[pl.ds(t * LANES, LANES)] = ((v >> sb) << sh) | (v & mh)

        def out_slice(j):
            return out_hbm.at[pl.ds(base + j * CHUNK, CHUNK)]

        def gather_chunk(j, b):
            return pltpu.async_copy(
                table_hbm.at[pair_v.at[pl.ds(j * CHUNK, CHUNK)]],
                buf_v.at[b],
                gsem,
            )

        gather_chunk(0, 0)

        def chunk_body(j, carry):
            b = j % 2
            nb = (j + 1) % 2

            @pl.when(j >= 2)
            def _():
                pltpu.make_async_copy(
                    obuf_v.at[b], out_slice(j - 2), osem
                ).wait()

            @pl.when(j + 1 < chunks_per_w)
            def _():
                gather_chunk(j + 1, nb)

            pltpu.make_async_copy(
                table_hbm.at[pair_v.at[pl.ds(j * CHUNK, CHUNK)]],
                buf_v.at[b],
                gsem,
            ).wait()

            @plsc.parallel_loop(0, CHUNK, unroll=4)
            def row_body(r):
                xs = [
                    buf_v[b, r, pl.ds(k * LANES, LANES)]
                    for k in range(2 * n_seg)
                ]
                odd = (
                    (
                        _lane_broadcast_i32(idx_v, j * CHUNK + r)
                        >> (BLOCK_I.bit_length() - 2)
                    )
                    & 1
                ) != 0
                hs = [
                    jnp.where(odd, xs[k + n_seg], xs[k]) for k in range(n_seg)
                ]
                sq = hs[0] * hs[0]
                for k in range(1, n_seg):
                    sq = sq + hs[k] * hs[k]
                for k in (1, 2, 4, 8):
                    sq = sq + _lane_shuffle(
                        sq, jnp.bitwise_xor(lax.iota(jnp.int32, LANES), k)
                    )
                y = _rsqrt_newton(sq)
                for k in range(n_seg):
                    obuf_v[b, r, pl.ds(k * LANES, LANES)] = hs[k] * y

            pltpu.async_copy(obuf_v.at[b], out_slice(j), osem)
            return carry

        lax.fori_loop(0, chunks_per_w, chunk_body, 0)
        pltpu.make_async_copy(
            obuf_v.at[(chunks_per_w - 2) % 2],
            out_slice(chunks_per_w - 2),
            osem,
        ).wait()
        pltpu.make_async_copy(
            obuf_v.at[(chunks_per_w - 1) % 2],
            out_slice(chunks_per_w - 1),
            osem,
        ).wait()

    return sc_kernel


def kernel(pointsUV, landmarks):
    n_points, d = pointsUV.shape
    b = landmarks.shape[0]
    per_w_chunk = NW * CHUNK
    chunks_per_w = -(-b // per_w_chunk)
    b_pad = chunks_per_w * per_w_chunk
    pad = (jnp.arange(b_pad - b, dtype=jnp.int32) * 8191) % n_points
    idx = jnp.concatenate([landmarks, pad]).reshape(NW, chunks_per_w * CHUNK)
    table2 = _make_tc_repack(n_points, d)(pointsUV.T)
    out = _make_sc_kernel(table2.shape[0], d, chunks_per_w)(table2, idx)
    return out[:b]
